# baseline (speedup 1.0000x reference)
import functools

import jax
import jax.numpy as jnp
from jax import lax
from jax.experimental import pallas as pl
from jax.experimental.pallas import tpu as pltpu
from jax.experimental.pallas import tpu_sc as plsc

_H = 768
_E = 64
_B = 32768
_HB = _B // 2
_BLK = 2048


def _dotnt(a, b):
    return lax.dot_general(a, b, (((1,), (1,)), ((), ())),
                           preferred_element_type=jnp.float32)


def _main_body(x_ref, wv_ref, wo_ref, rw1_ref, rw2_ref, bv_ref, bo_ref,
               logits_ref, load_ref, ent_ref,
               ls_ref, load_acc, ent_acc,
               wv16_ref, wo16_ref, rw116_ref, rw216_ref):
    i = pl.program_id(0)
    nblk = pl.num_programs(0) - 1
    first = i == 0

    @pl.when(first)
    def _prep_weights():
        wv16_ref[...] = wv_ref[0].astype(jnp.bfloat16)
        wo16_ref[...] = wo_ref[...].astype(jnp.bfloat16)
        rw116_ref[...] = rw1_ref[...].astype(jnp.bfloat16)
        rw216_ref[...] = rw2_ref[...].astype(jnp.bfloat16)

    l = ls_ref[...]
    m1 = jnp.max(l, axis=-1, keepdims=True)
    d = l - m1
    e = jnp.exp(d)
    s = jnp.sum(e, axis=-1, keepdims=True)
    rs = 1.0 / s
    p = e * rs

    load_c = jnp.where(first, 0.0, jnp.sum(p, axis=0, keepdims=True))
    load_acc[...] = jnp.where(first, 0.0, load_acc[...]) + load_c
    plogp = jnp.sum(e * d, axis=-1, keepdims=True) * rs - jnp.log(s)
    ent_c = jnp.where(first, 0.0, jnp.sum(plogp, axis=0, keepdims=True))
    ent_acc[...] = jnp.where(first, 0.0, ent_acc[...]) + ent_c

    x16 = x_ref[...].astype(jnp.bfloat16)
    v = _dotnt(x16, wv16_ref[...]) + bv_ref[0]
    a = _dotnt(v.astype(jnp.bfloat16), wo16_ref[...]) + bo_ref[...]
    logits = _dotnt(x16, rw116_ref[...]) + _dotnt(a.astype(jnp.bfloat16),
                                                  rw216_ref[...])
    logits_ref[...] = logits
    ls_ref[...] = logits

    @pl.when(i == nblk)
    def _emit_partials():
        load_ref[...] = load_acc[...]
        ent_ref[...] = ent_acc[...]


def _tc_part(hidden_states, in_proj_w, in_proj_b, out_proj_w, out_proj_b,
             router_w, half):
    ipw3 = in_proj_w.reshape(3, _H, _H)
    ipb3 = in_proj_b.reshape(3, 1, _H)
    bo2 = out_proj_b.reshape(1, _H)

    nblk = _HB // _BLK
    base = half * nblk
    logits, load, ent = pl.pallas_call(
        _main_body,
        grid=(nblk + 1,),
        in_specs=[
            pl.BlockSpec((_BLK, _H),
                         lambda i: (base + jnp.minimum(i, nblk - 1), 0)),
            pl.BlockSpec((1, _H, _H), lambda i: (2, 0, 0)),
            pl.BlockSpec((_H, _H), lambda i: (0, 0)),
            pl.BlockSpec((_E, _H), lambda i: (0, 0)),
            pl.BlockSpec((_E, _H), lambda i: (0, 1)),
            pl.BlockSpec((1, 1, _H), lambda i: (2, 0, 0)),
            pl.BlockSpec((1, _H), lambda i: (0, 0)),
        ],
        out_specs=[
            pl.BlockSpec((_BLK, _E), lambda i: (jnp.minimum(i, nblk - 1), 0)),
            pl.BlockSpec((1, _E), lambda i: (0, 0)),
            pl.BlockSpec((1, 1), lambda i: (0, 0)),
        ],
        out_shape=[
            jax.ShapeDtypeStruct((_HB, _E), jnp.float32),
            jax.ShapeDtypeStruct((1, _E), jnp.float32),
            jax.ShapeDtypeStruct((1, 1), jnp.float32),
        ],
        scratch_shapes=[
            pltpu.VMEM((_BLK, _E), jnp.float32),
            pltpu.VMEM((1, _E), jnp.float32),
            pltpu.VMEM((1, 1), jnp.float32),
            pltpu.VMEM((_H, _H), jnp.bfloat16),
            pltpu.VMEM((_H, _H), jnp.bfloat16),
            pltpu.VMEM((_E, _H), jnp.bfloat16),
            pltpu.VMEM((_E, _H), jnp.bfloat16),
        ],
        compiler_params=pltpu.CompilerParams(
            dimension_semantics=("arbitrary",)),
    )(hidden_states, ipw3, out_proj_w, router_w, router_w, ipb3, bo2)
    return logits, load, ent


_L = 16
_CH = 256


def _sc_body(logits_hbm, i1_hbm, i2_hbm, w1_hbm, w2_hbm,
             s0, s1, i1b, i2b, w1b, w2b, sem0, sem1):
    nc = 2
    wid = lax.axis_index("s") * nc + lax.axis_index("c")
    rpw = _HB // 32
    nch = rpw // _CH
    base = wid * rpw
    lanes = lax.iota(jnp.int32, _L)
    stages = (s0, s1)
    sems = (sem0, sem1)

    def make_tile(stage, c):
        def tile(t, carry2):
            rloc = lanes + t * _L
            m1 = jnp.full((_L,), -jnp.inf, jnp.float32)
            m2 = jnp.full((_L,), -jnp.inf, jnp.float32)
            i1 = jnp.zeros((_L,), jnp.int32)
            i2 = jnp.zeros((_L,), jnp.int32)
            for e in range(_E):
                col = plsc.load_gather(
                    stage, [rloc, jnp.full((_L,), e, jnp.int32)])
                gt = col > m1
                c2 = col > m2
                ev = jnp.full((_L,), e, jnp.int32)
                m2 = jnp.where(gt, m1, jnp.where(c2, col, m2))
                i2 = jnp.where(gt, i1, jnp.where(c2, ev, i2))
                m1 = jnp.where(gt, col, m1)
                i1 = jnp.where(gt, ev, i1)
            off = c * _CH + t * _L
            i1b[pl.ds(off, _L)] = i1
            i2b[pl.ds(off, _L)] = i2
            w1 = 1.0 / (1.0 + jnp.exp(m2 - m1))
            w1b[pl.ds(off, _L)] = w1
            w2b[pl.ds(off, _L)] = 1.0 - w1
            return carry2
        return tile

    pltpu.make_async_copy(logits_hbm.at[pl.ds(base, _CH)], s0, sem0).start()
    for c in range(nch):
        cur = c % 2
        pltpu.make_async_copy(
            logits_hbm.at[pl.ds(base + c * _CH, _CH)], stages[cur],
            sems[cur]).wait()
        if c + 1 < nch:
            nxt = (c + 1) % 2
            pltpu.make_async_copy(
                logits_hbm.at[pl.ds(base + (c + 1) * _CH, _CH)],
                stages[nxt], sems[nxt]).start()
        lax.fori_loop(0, _CH // _L, make_tile(stages[cur], c), 0)

    pltpu.sync_copy(i1b, i1_hbm.at[pl.ds(base, rpw)])
    pltpu.sync_copy(i2b, i2_hbm.at[pl.ds(base, rpw)])
    pltpu.sync_copy(w1b, w1_hbm.at[pl.ds(base, rpw)])
    pltpu.sync_copy(w2b, w2_hbm.at[pl.ds(base, rpw)])


def _sc_part(logits):
    rpw = _HB // 32
    mesh = plsc.VectorSubcoreMesh(core_axis_name="c", subcore_axis_name="s")
    k = functools.partial(
        pl.kernel,
        mesh=mesh,
        out_type=[jax.ShapeDtypeStruct((_HB,), jnp.int32),
                  jax.ShapeDtypeStruct((_HB,), jnp.int32),
                  jax.ShapeDtypeStruct((_HB,), jnp.float32),
                  jax.ShapeDtypeStruct((_HB,), jnp.float32)],
        scratch_types=[pltpu.VMEM((_CH, _E), jnp.float32),
                       pltpu.VMEM((_CH, _E), jnp.float32),
                       pltpu.VMEM((rpw,), jnp.int32),
                       pltpu.VMEM((rpw,), jnp.int32),
                       pltpu.VMEM((rpw,), jnp.float32),
                       pltpu.VMEM((rpw,), jnp.float32),
                       pltpu.SemaphoreType.DMA,
                       pltpu.SemaphoreType.DMA],
        compiler_params=pltpu.CompilerParams(needs_layout_passes=False),
    )(_sc_body)
    return k(logits)


def kernel(hidden_states, enc_w1, enc_b1, ln_g, ln_b, enc_w2, enc_b2,
           in_proj_w, in_proj_b, out_proj_w, out_proj_b, router_w):
    l0, load0, ent0 = _tc_part(hidden_states, in_proj_w, in_proj_b,
                               out_proj_w, out_proj_b, router_w, 0)
    sc0 = _sc_part(l0)
    l1, load1, ent1 = _tc_part(hidden_states, in_proj_w, in_proj_b,
                               out_proj_w, out_proj_b, router_w, 1)
    sc1 = _sc_part(l1)

    logits = jnp.concatenate([l0, l1], axis=0)
    idx = jnp.stack([jnp.concatenate([sc0[0], sc1[0]]),
                     jnp.concatenate([sc0[1], sc1[1]])], axis=1)
    w = jnp.stack([jnp.concatenate([sc0[2], sc1[2]]),
                   jnp.concatenate([sc0[3], sc1[3]])], axis=1)

    el = (load0 + load1) / _B
    mu = jnp.mean(el)
    lv = jnp.sum((el - mu) ** 2) / (_E - 1)
    ent = -(ent0 + ent1).reshape(()) / _B
    return (logits, idx, w, lv, ent)


# speedup vs baseline: 1.0491x; 1.0491x over previous
import functools

import jax
import jax.numpy as jnp
from jax import lax
from jax.experimental import pallas as pl
from jax.experimental.pallas import tpu as pltpu
from jax.experimental.pallas import tpu_sc as plsc

_H = 768
_E = 64
_B = 32768
_BLK = 2048


def _dotnt(a, b):
    return lax.dot_general(a, b, (((1,), (1,)), ((), ())),
                           preferred_element_type=jnp.float32)


def _main_body(x_ref, wv_ref, wo_ref, rw1_ref, rw2_ref, bv_ref, bo_ref,
               logits_ref, lv_ref, ent_ref,
               ls_ref, load_acc, ent_acc,
               wv16_ref, wo16_ref, rw116_ref, rw216_ref):
    i = pl.program_id(0)
    nblk = pl.num_programs(0) - 1
    first = i == 0

    @pl.when(first)
    def _prep_weights():
        wv16_ref[...] = wv_ref[0].astype(jnp.bfloat16)
        wo16_ref[...] = wo_ref[...].astype(jnp.bfloat16)
        rw116_ref[...] = rw1_ref[...].astype(jnp.bfloat16)
        rw216_ref[...] = rw2_ref[...].astype(jnp.bfloat16)

    l = ls_ref[...]
    m1 = jnp.max(l, axis=-1, keepdims=True)
    d = l - m1
    e = jnp.exp(d)
    s = jnp.sum(e, axis=-1, keepdims=True)
    rs = 1.0 / s
    p = e * rs

    load_c = jnp.where(first, 0.0, jnp.sum(p, axis=0, keepdims=True))
    load_acc[...] = jnp.where(first, 0.0, load_acc[...]) + load_c
    plogp = jnp.sum(e * d, axis=-1, keepdims=True) * rs - jnp.log(s)
    ent_c = jnp.where(first, 0.0, jnp.sum(plogp, axis=0, keepdims=True))
    ent_acc[...] = jnp.where(first, 0.0, ent_acc[...]) + ent_c

    x16 = x_ref[...].astype(jnp.bfloat16)
    v = _dotnt(x16, wv16_ref[...]) + bv_ref[0]
    a = _dotnt(v.astype(jnp.bfloat16), wo16_ref[...]) + bo_ref[...]
    logits = _dotnt(x16, rw116_ref[...]) + _dotnt(a.astype(jnp.bfloat16),
                                                  rw216_ref[...])
    logits_ref[...] = logits
    ls_ref[...] = logits

    @pl.when(i == nblk)
    def _finalize():
        el = load_acc[...] / _B
        mu = jnp.mean(el)
        lv_ref[...] = jnp.sum((el - mu) ** 2, keepdims=True)[:, :1] / (_E - 1)
        ent_ref[...] = -ent_acc[...] / _B


def _tc_part(hidden_states, in_proj_w, in_proj_b, out_proj_w, out_proj_b,
             router_w):
    ipw3 = in_proj_w.reshape(3, _H, _H)
    ipb3 = in_proj_b.reshape(3, 1, _H)
    bo2 = out_proj_b.reshape(1, _H)

    nblk = _B // _BLK
    logits, lv, ent = pl.pallas_call(
        _main_body,
        grid=(nblk + 1,),
        in_specs=[
            pl.BlockSpec((_BLK, _H), lambda i: (jnp.minimum(i, nblk - 1), 0)),
            pl.BlockSpec((1, _H, _H), lambda i: (2, 0, 0)),
            pl.BlockSpec((_H, _H), lambda i: (0, 0)),
            pl.BlockSpec((_E, _H), lambda i: (0, 0)),
            pl.BlockSpec((_E, _H), lambda i: (0, 1)),
            pl.BlockSpec((1, 1, _H), lambda i: (2, 0, 0)),
            pl.BlockSpec((1, _H), lambda i: (0, 0)),
        ],
        out_specs=[
            pl.BlockSpec((_BLK, _E), lambda i: (jnp.minimum(i, nblk - 1), 0)),
            pl.BlockSpec((1, 1), lambda i: (0, 0)),
            pl.BlockSpec((1, 1), lambda i: (0, 0)),
        ],
        out_shape=[
            jax.ShapeDtypeStruct((_B, _E), jnp.float32),
            jax.ShapeDtypeStruct((1, 1), jnp.float32),
            jax.ShapeDtypeStruct((1, 1), jnp.float32),
        ],
        scratch_shapes=[
            pltpu.VMEM((_BLK, _E), jnp.float32),
            pltpu.VMEM((1, _E), jnp.float32),
            pltpu.VMEM((1, 1), jnp.float32),
            pltpu.VMEM((_H, _H), jnp.bfloat16),
            pltpu.VMEM((_H, _H), jnp.bfloat16),
            pltpu.VMEM((_E, _H), jnp.bfloat16),
            pltpu.VMEM((_E, _H), jnp.bfloat16),
        ],
        compiler_params=pltpu.CompilerParams(
            dimension_semantics=("arbitrary",)),
    )(hidden_states, ipw3, out_proj_w, router_w, router_w, ipb3, bo2)
    return logits, lv, ent


_L = 16
_CH = 256


def _sc_body(logits_hbm, i1_hbm, i2_hbm, w1_hbm, w2_hbm,
             s0, s1, i1b, i2b, w1b, w2b, sem0, sem1):
    nc = 2
    wid = lax.axis_index("s") * nc + lax.axis_index("c")
    rpw = _B // 32
    nch = rpw // _CH
    base = wid * rpw
    lanes = lax.iota(jnp.int32, _L)
    stages = (s0, s1)
    sems = (sem0, sem1)

    def make_tile(stage, c):
        def tile(t, carry2):
            g = [dict(r=lanes + (2 * t + j) * _L,
                      m1=jnp.full((_L,), -jnp.inf, jnp.float32),
                      m2=jnp.full((_L,), -jnp.inf, jnp.float32),
                      i1=jnp.zeros((_L,), jnp.int32),
                      i2=jnp.zeros((_L,), jnp.int32)) for j in range(2)]
            for e in range(_E):
                ev = jnp.full((_L,), e, jnp.int32)
                ecol = jnp.full((_L,), e, jnp.int32)
                for s in g:
                    col = plsc.load_gather(stage, [s["r"], ecol])
                    gt = col > s["m1"]
                    c2 = col > s["m2"]
                    s["m2"] = jnp.where(gt, s["m1"],
                                        jnp.where(c2, col, s["m2"]))
                    s["i2"] = jnp.where(gt, s["i1"],
                                        jnp.where(c2, ev, s["i2"]))
                    s["m1"] = jnp.where(gt, col, s["m1"])
                    s["i1"] = jnp.where(gt, ev, s["i1"])
            for j, s in enumerate(g):
                off = c * _CH + (2 * t + j) * _L
                i1b[pl.ds(off, _L)] = s["i1"]
                i2b[pl.ds(off, _L)] = s["i2"]
                w1 = 1.0 / (1.0 + jnp.exp(s["m2"] - s["m1"]))
                w1b[pl.ds(off, _L)] = w1
                w2b[pl.ds(off, _L)] = 1.0 - w1
            return carry2
        return tile

    h = pltpu.make_async_copy(logits_hbm.at[pl.ds(base, _CH)], s0, sem0)
    h.start()
    for c in range(nch):
        cur = c % 2
        pltpu.make_async_copy(
            logits_hbm.at[pl.ds(base + c * _CH, _CH)], stages[cur],
            sems[cur]).wait()
        if c + 1 < nch:
            nxt = (c + 1) % 2
            pltpu.make_async_copy(
                logits_hbm.at[pl.ds(base + (c + 1) * _CH, _CH)],
                stages[nxt], sems[nxt]).start()
        lax.fori_loop(0, _CH // (2 * _L), make_tile(stages[cur], c), 0)

    pltpu.sync_copy(i1b, i1_hbm.at[pl.ds(base, rpw)])
    pltpu.sync_copy(i2b, i2_hbm.at[pl.ds(base, rpw)])
    pltpu.sync_copy(w1b, w1_hbm.at[pl.ds(base, rpw)])
    pltpu.sync_copy(w2b, w2_hbm.at[pl.ds(base, rpw)])


def _sc_part(logits):
    rpw = _B // 32
    mesh = plsc.VectorSubcoreMesh(core_axis_name="c", subcore_axis_name="s")
    k = functools.partial(
        pl.kernel,
        mesh=mesh,
        out_type=[jax.ShapeDtypeStruct((_B,), jnp.int32),
                  jax.ShapeDtypeStruct((_B,), jnp.int32),
                  jax.ShapeDtypeStruct((_B,), jnp.float32),
                  jax.ShapeDtypeStruct((_B,), jnp.float32)],
        scratch_types=[pltpu.VMEM((_CH, _E), jnp.float32),
                       pltpu.VMEM((_CH, _E), jnp.float32),
                       pltpu.VMEM((rpw,), jnp.int32),
                       pltpu.VMEM((rpw,), jnp.int32),
                       pltpu.VMEM((rpw,), jnp.float32),
                       pltpu.VMEM((rpw,), jnp.float32),
                       pltpu.SemaphoreType.DMA,
                       pltpu.SemaphoreType.DMA],
        compiler_params=pltpu.CompilerParams(needs_layout_passes=False),
    )(_sc_body)
    return k(logits)


def kernel(hidden_states, enc_w1, enc_b1, ln_g, ln_b, enc_w2, enc_b2,
           in_proj_w, in_proj_b, out_proj_w, out_proj_b, router_w):
    logits, lv, ent = _tc_part(hidden_states, in_proj_w, in_proj_b,
                               out_proj_w, out_proj_b, router_w)
    i1, i2, w1, w2 = _sc_part(logits)
    idx = jnp.stack([i1, i2], axis=1)
    w = jnp.stack([w1, w2], axis=1)
    return (logits, idx, w, lv.reshape(()), ent.reshape(()))
